# baseline (speedup 1.0000x reference)
import functools

import jax
import jax.numpy as jnp
from jax import lax
from jax.experimental import pallas as pl
from jax.experimental.pallas import tpu as pltpu
from jax.experimental.pallas import tpu_sc as plsc

NC = 2
NS = 16
NW = NC * NS

DIM = 32
LG = 128 // DIM
IDX_W = 128
GROUP = 5
NBUF = 4

TAB_BLK = 2048
VQP = 123 * TAB_BLK


def _eye():
  return jnp.eye(DIM, dtype=jnp.float32)


def _t(x):
  return lax.dot_general(
      _eye(), x, (((1,), (1,)), ((), ())),
      precision=lax.Precision.HIGHEST,
      preferred_element_type=jnp.float32)


def _pack_table(table_t):
  def body(*refs):
    in_refs, o_ref = refs[:LG], refs[LG]
    o_ref[...] = jnp.concatenate(
        [lax.dot_general(
            r[...], _eye(), (((0,), (0,)), ((), ())),
            precision=lax.Precision.HIGHEST,
            preferred_element_type=jnp.float32) for r in in_refs],
        axis=1)

  nblk = VQP // TAB_BLK
  last = (table_t.shape[1] - 1) // TAB_BLK

  return pl.pallas_call(
      body,
      out_shape=jax.ShapeDtypeStruct((VQP, LG * DIM), jnp.float32),
      grid=(nblk,),
      in_specs=[
          pl.BlockSpec(
              (DIM, TAB_BLK),
              functools.partial(
                  lambda j, c: (0, jnp.minimum(j * nblk + c, last)), j))
          for j in range(LG)
      ],
      out_specs=pl.BlockSpec((TAB_BLK, LG * DIM), lambda c: (c, 0)),
  )(*([table_t] * LG))


def _unpack_out(g2):
  l, bq, _ = g2.shape
  b = bq * LG

  def body(g_ref, o_ref):
    g = g_ref[0]
    o_ref[0] = jnp.concatenate(
        [_t(g[:, q * DIM:(q + 1) * DIM]) for q in range(LG)], axis=1)

  return pl.pallas_call(
      body,
      out_shape=jax.ShapeDtypeStruct((l, DIM, b), jnp.float32),
      grid=(l,),
      in_specs=[pl.BlockSpec((1, bq, LG * DIM), lambda i: (i, 0, 0))],
      out_specs=pl.BlockSpec((1, DIM, b), lambda i: (i, 0, 0)),
  )(g2)


def _make_gather(n_total: int):
  rows_per_w = n_total // NW
  idx_rows_w = rows_per_w // IDX_W
  n_trips = idx_rows_w // GROUP
  chunk = GROUP * IDX_W
  assert n_trips % NBUF == 0 and n_trips >= 2 * NBUF

  mesh = plsc.VectorSubcoreMesh(
      core_axis_name="c", subcore_axis_name="s", num_cores=NC,
      num_subcores=NS)

  @functools.partial(
      pl.kernel,
      out_type=jax.ShapeDtypeStruct((n_total, DIM), jnp.float32),
      mesh=mesh,
      scratch_types=[
          pltpu.VMEM((idx_rows_w, IDX_W), jnp.int32),
          [pltpu.VMEM((chunk, DIM), jnp.float32) for _ in range(NBUF)],
          [pltpu.SemaphoreType.DMA for _ in range(NBUF)],
          [pltpu.SemaphoreType.DMA for _ in range(NBUF)],
      ],
      compiler_params=pltpu.CompilerParams(use_tc_tiling_on_sc=False),
  )
  def gather_kernel(table_hbm, idx_hbm, out_hbm, idx_v, bufs, sg, sw):
    wid = lax.axis_index("s") * NC + lax.axis_index("c")
    idx_row_base = wid * idx_rows_w
    out_base = wid * rows_per_w

    pltpu.sync_copy(idx_hbm.at[pl.ds(idx_row_base, idx_rows_w)], idx_v)

    def issue_g(t, s):
      for g in range(GROUP):
        pltpu.async_copy(
            table_hbm.at[idx_v.at[t * GROUP + g]],
            bufs[s].at[pl.ds(g * IDX_W, IDX_W)],
            sg[s],
        )

    def wait_g(s):
      pltpu.make_async_copy(
          table_hbm.at[pl.ds(0, chunk)], bufs[s], sg[s]).wait()

    def issue_w(t, s):
      pltpu.async_copy(
          bufs[s], out_hbm.at[pl.ds(out_base + t * chunk, chunk)], sw[s])

    def wait_w(s):
      pltpu.make_async_copy(
          bufs[s], out_hbm.at[pl.ds(0, chunk)], sw[s]).wait()

    for s in range(NBUF):
      issue_g(s, s)
      if s >= 1:
        wait_g(s - 1)
        issue_w(s - 1, s - 1)

    def outer(o_idx, _):
      o = o_idx * NBUF
      for s in range(NBUF):
        t = o + s
        wait_w(s)
        issue_g(t, s)
        ps = (s - 1) % NBUF
        wait_g(ps)
        issue_w(t - 1, ps)
      return _

    lax.fori_loop(1, n_trips // NBUF, outer, None)

    wait_g(NBUF - 1)
    issue_w(n_trips - 1, NBUF - 1)
    for s in range(NBUF):
      wait_w(s)

  return gather_kernel


def kernel(x, table):
  b, l = x.shape
  n_total = b * l

  packed = _pack_table(table.T)
  rows_view = packed.reshape(VQP * LG, DIM)

  xi = (x % VQP) * LG + x // VQP
  x3 = jnp.swapaxes(xi, 0, 1).reshape(l, LG, b // LG)
  idx_t = jnp.swapaxes(x3, 1, 2).reshape(n_total // IDX_W, IDX_W)

  gathered = _make_gather(n_total)(rows_view, idx_t)
  out3 = _unpack_out(gathered.reshape(l, b // LG, LG * DIM))
  return jnp.transpose(out3, (2, 0, 1))


# speedup vs baseline: 1.2661x; 1.2661x over previous
import functools

import jax
import jax.numpy as jnp
from jax import lax
from jax.experimental import pallas as pl
from jax.experimental.pallas import tpu as pltpu
from jax.experimental.pallas import tpu_sc as plsc

NC = 2
NS = 16
NW = NC * NS

DIM = 32
IDX_W = 128
GROUP = 5
NBUF = 4


def _make_gather(n_total: int):
  rows_per_w = n_total // NW
  idx_rows_w = rows_per_w // IDX_W
  n_trips = idx_rows_w // GROUP
  chunk = GROUP * IDX_W
  assert n_trips % NBUF == 0 and n_trips >= 2 * NBUF

  mesh = plsc.VectorSubcoreMesh(
      core_axis_name="c", subcore_axis_name="s", num_cores=NC,
      num_subcores=NS)

  @functools.partial(
      pl.kernel,
      out_type=jax.ShapeDtypeStruct((n_total, DIM), jnp.float32),
      mesh=mesh,
      scratch_types=[
          pltpu.VMEM((idx_rows_w, IDX_W), jnp.int32),
          [pltpu.VMEM((chunk, DIM), jnp.float32) for _ in range(NBUF)],
          [pltpu.SemaphoreType.DMA for _ in range(NBUF)],
          [pltpu.SemaphoreType.DMA for _ in range(NBUF)],
      ],
      compiler_params=pltpu.CompilerParams(use_tc_tiling_on_sc=False),
  )
  def gather_kernel(table_hbm, idx_hbm, out_hbm, idx_v, bufs, sg, sw):
    wid = lax.axis_index("s") * NC + lax.axis_index("c")
    idx_row_base = wid * idx_rows_w
    out_base = wid * rows_per_w

    pltpu.sync_copy(idx_hbm.at[pl.ds(idx_row_base, idx_rows_w)], idx_v)

    def issue_g(t, s):
      for g in range(GROUP):
        pltpu.async_copy(
            table_hbm.at[idx_v.at[t * GROUP + g]],
            bufs[s].at[pl.ds(g * IDX_W, IDX_W)],
            sg[s],
        )

    def wait_g(s):
      pltpu.make_async_copy(
          table_hbm.at[pl.ds(0, chunk)], bufs[s], sg[s]).wait()

    def issue_w(t, s):
      pltpu.async_copy(
          bufs[s], out_hbm.at[pl.ds(out_base + t * chunk, chunk)], sw[s])

    def wait_w(s):
      pltpu.make_async_copy(
          bufs[s], out_hbm.at[pl.ds(0, chunk)], sw[s]).wait()

    for s in range(NBUF):
      issue_g(s, s)
      if s >= 1:
        wait_g(s - 1)
        issue_w(s - 1, s - 1)

    def outer(o_idx, _):
      o = o_idx * NBUF
      for s in range(NBUF):
        t = o + s
        wait_w(s)
        issue_g(t, s)
        ps = (s - 1) % NBUF
        wait_g(ps)
        issue_w(t - 1, ps)
      return _

    lax.fori_loop(1, n_trips // NBUF, outer, None)

    wait_g(NBUF - 1)
    issue_w(n_trips - 1, NBUF - 1)
    for s in range(NBUF):
      wait_w(s)

  return gather_kernel


def kernel(x, table):
  b, l = x.shape
  n_total = b * l
  idx_t = jnp.swapaxes(x, 0, 1).reshape(n_total // IDX_W, IDX_W)
  gathered = _make_gather(n_total)(table, idx_t)
  return jnp.transpose(gathered.reshape(l, b, DIM), (1, 0, 2))
